# baseline (speedup 1.0000x reference)
import functools

import jax
import jax.numpy as jnp
from jax.experimental import pallas as pl
from jax.experimental.pallas import tpu as pltpu
from jax.experimental.pallas import tpu_sc as plsc

B = 16384
GMF_D = 32
MLP_D = 128
_W = 128


@functools.cache
def _sc_gather_fn():
    mesh = plsc.VectorSubcoreMesh(core_axis_name="core",
                                  subcore_axis_name="subcore")

    @functools.partial(
        pl.kernel,
        out_type=(
            jax.ShapeDtypeStruct((B, MLP_D), jnp.float32),
            jax.ShapeDtypeStruct((B, MLP_D), jnp.float32),
            jax.ShapeDtypeStruct((B, MLP_D), jnp.float32),
            jax.ShapeDtypeStruct((B, MLP_D), jnp.float32),
        ),
        mesh=mesh,
    )
    def _sc_gather(uidx_hbm, iidx_hbm, tu_hbm, ti_hbm, eu_gmf_hbm, ei_gmf_hbm,
                   eu_mlp_hbm, ei_mlp_hbm, gu_hbm, gi_hbm, mu_hbm, mi_hbm):
        idx_spec = pl.BlockSpec((1, _W), lambda i: (0, i))
        row_spec = pl.BlockSpec((_W, MLP_D), lambda i: (i, 0))

        def pair_pipeline(table_a, table_b):
            def body(a_v, b_v, out_a_v, out_b_v):
                pltpu.sync_copy(table_a.at[a_v.at[0]], out_a_v)
                pltpu.sync_copy(table_b.at[b_v.at[0]], out_b_v)

            return pltpu.emit_pipeline(
                body,
                grid=(B // _W,),
                in_specs=[idx_spec, idx_spec],
                out_specs=[row_spec, row_spec],
                core_axis_name=("core", "subcore"),
                dimension_semantics=(pltpu.PARALLEL,),
            )

        pair_pipeline(eu_mlp_hbm, ei_mlp_hbm)(uidx_hbm, iidx_hbm,
                                              mu_hbm, mi_hbm)
        pair_pipeline(eu_gmf_hbm, ei_gmf_hbm)(tu_hbm, ti_hbm,
                                              gu_hbm, gi_hbm)

    return _sc_gather


_BLK = 2048


def _tc_body(guw, giw, mu, mi, su, si, w0u, w0i, b0, w1, b1, w2, b2, wpg, wpm,
             bp, out):
    h = jnp.dot(mu[...], w0u[...], preferred_element_type=jnp.float32)
    h = h + jnp.dot(mi[...], w0i[...], preferred_element_type=jnp.float32)
    h = jnp.maximum(h + b0[...], 0.0)
    h = jnp.maximum(
        jnp.dot(h, w1[...], preferred_element_type=jnp.float32) + b1[...], 0.0)
    h = jnp.maximum(
        jnp.dot(h, w2[...], preferred_element_type=jnp.float32) + b2[...], 0.0)

    def sel(wide, s):
        r = jnp.where(s == 0, wide[:, 0:32], 0.0)
        r = r + jnp.where(s == 1, wide[:, 32:64], 0.0)
        r = r + jnp.where(s == 2, wide[:, 64:96], 0.0)
        return r + jnp.where(s == 3, wide[:, 96:128], 0.0)

    g = sel(guw[...], su[...]) * sel(giw[...], si[...])
    pred = (jnp.sum(g * wpg[...], axis=1) + jnp.sum(h * wpm[...], axis=1)
            + bp[0, 0])
    out[...] = pred


def _tc_fuse(guw, giw, mu, mi, su, si, w0u, w0i, b0, w1, b1, w2, b2, wpg, wpm,
             bp):
    n_blk = B // _BLK
    batch_spec = lambda d: pl.BlockSpec((_BLK, d), lambda i: (i, 0))
    full = lambda a: pl.BlockSpec(a.shape, lambda i: (0,) * a.ndim)
    return pl.pallas_call(
        _tc_body,
        grid=(n_blk,),
        in_specs=[
            batch_spec(MLP_D), batch_spec(MLP_D),
            batch_spec(MLP_D), batch_spec(MLP_D),
            batch_spec(1), batch_spec(1),
            full(w0u), full(w0i), full(b0), full(w1), full(b1),
            full(w2), full(b2), full(wpg), full(wpm), full(bp),
        ],
        out_specs=pl.BlockSpec((_BLK,), lambda i: (i,)),
        out_shape=jax.ShapeDtypeStruct((B,), jnp.float32),
    )(guw, giw, mu, mi, su, si, w0u, w0i, b0, w1, b1, w2, b2, wpg, wpm, bp)


def kernel(x, eu_gmf, ei_gmf, eu_mlp, ei_mlp, W0, b0, W1, b1, W2, b2, Wp, bp):
    uidx = x[:, 0]
    iidx = x[:, 1]
    n_pack = 128 // GMF_D
    eu_gmf_w = eu_gmf.reshape(eu_gmf.shape[0] // n_pack, 128)
    ei_gmf_w = ei_gmf.reshape(ei_gmf.shape[0] // n_pack, 128)
    guw, giw, mu, mi = _sc_gather_fn()(
        uidx.reshape(1, B), iidx.reshape(1, B),
        (uidx // n_pack).reshape(1, B), (iidx // n_pack).reshape(1, B),
        eu_gmf_w, ei_gmf_w, eu_mlp, ei_mlp)
    pred = _tc_fuse(
        guw, giw, mu, mi,
        (uidx % n_pack).reshape(B, 1), (iidx % n_pack).reshape(B, 1),
        W0[:, :MLP_D].T, W0[:, MLP_D:].T, b0.reshape(1, -1),
        W1.T, b1.reshape(1, -1), W2.T, b2.reshape(1, -1),
        Wp[:, :GMF_D], Wp[:, GMF_D:], bp.reshape(1, 1),
    )
    return pred


# speedup vs baseline: 1.0040x; 1.0040x over previous
import functools

import jax
import jax.numpy as jnp
from jax.experimental import pallas as pl
from jax.experimental.pallas import tpu as pltpu
from jax.experimental.pallas import tpu_sc as plsc

B = 16384
GMF_D = 32
MLP_D = 128
_W = 128


@functools.cache
def _sc_gather_fn():
    mesh = plsc.VectorSubcoreMesh(core_axis_name="core",
                                  subcore_axis_name="subcore")

    @functools.partial(
        pl.kernel,
        out_type=(
            jax.ShapeDtypeStruct((B, MLP_D), jnp.float32),
            jax.ShapeDtypeStruct((B, MLP_D), jnp.float32),
            jax.ShapeDtypeStruct((B, MLP_D), jnp.float32),
            jax.ShapeDtypeStruct((B, MLP_D), jnp.float32),
        ),
        mesh=mesh,
        scratch_types=[pltpu.SemaphoreType.DMA] * 2,
    )
    def _sc_gather(uidx_hbm, iidx_hbm, tu_hbm, ti_hbm, eu_gmf_hbm, ei_gmf_hbm,
                   eu_mlp_hbm, ei_mlp_hbm, gu_hbm, gi_hbm, mu_hbm, mi_hbm,
                   s0, s1):
        idx_spec = pl.BlockSpec((1, _W), lambda i: (0, i))
        row_spec = pl.BlockSpec((_W, MLP_D), lambda i: (i, 0))

        def pair_pipeline(table_a, table_b):
            def body(a_v, b_v, out_a_v, out_b_v):
                c0 = pltpu.make_async_copy(table_a.at[a_v.at[0]], out_a_v, s0)
                c1 = pltpu.make_async_copy(table_b.at[b_v.at[0]], out_b_v, s1)
                c0.start(); c1.start()
                c0.wait(); c1.wait()

            return pltpu.emit_pipeline(
                body,
                grid=(B // _W,),
                in_specs=[idx_spec, idx_spec],
                out_specs=[row_spec, row_spec],
                core_axis_name=("core", "subcore"),
                dimension_semantics=(pltpu.PARALLEL,),
            )

        pair_pipeline(eu_mlp_hbm, ei_mlp_hbm)(uidx_hbm, iidx_hbm,
                                              mu_hbm, mi_hbm)
        pair_pipeline(eu_gmf_hbm, ei_gmf_hbm)(tu_hbm, ti_hbm, gu_hbm, gi_hbm)

    return _sc_gather


_BLK = 2048


def _tc_body(guw, giw, mu, mi, su, si, w0u, w0i, b0, w1, b1, w2, b2, wpg, wpm,
             bp, out):
    mu16 = mu[...].astype(jnp.bfloat16)
    mi16 = mi[...].astype(jnp.bfloat16)
    h = jnp.dot(mu16, w0u[...].astype(jnp.bfloat16),
                preferred_element_type=jnp.float32)
    h = h + jnp.dot(mi16, w0i[...].astype(jnp.bfloat16),
                    preferred_element_type=jnp.float32)
    h = jnp.maximum(h + b0[...], 0.0)
    h = jnp.maximum(
        jnp.dot(h, w1[...], preferred_element_type=jnp.float32) + b1[...], 0.0)
    h = jnp.maximum(
        jnp.dot(h, w2[...], preferred_element_type=jnp.float32) + b2[...], 0.0)

    def sel(wide, s):
        r = jnp.where(s == 0, wide[:, 0:32], 0.0)
        r = r + jnp.where(s == 1, wide[:, 32:64], 0.0)
        r = r + jnp.where(s == 2, wide[:, 64:96], 0.0)
        return r + jnp.where(s == 3, wide[:, 96:128], 0.0)

    g = sel(guw[...], su[...]) * sel(giw[...], si[...])
    pred = (jnp.sum(g * wpg[...], axis=1) + jnp.sum(h * wpm[...], axis=1)
            + bp[0, 0])
    out[...] = pred


def _tc_fuse(guw, giw, mu, mi, su, si, w0u, w0i, b0, w1, b1, w2, b2, wpg, wpm,
             bp):
    n_blk = B // _BLK
    batch_spec = lambda d: pl.BlockSpec((_BLK, d), lambda i: (i, 0))
    full = lambda a: pl.BlockSpec(a.shape, lambda i: (0,) * a.ndim)
    return pl.pallas_call(
        _tc_body,
        grid=(n_blk,),
        in_specs=[
            batch_spec(MLP_D), batch_spec(MLP_D),
            batch_spec(MLP_D), batch_spec(MLP_D),
            batch_spec(1), batch_spec(1),
            full(w0u), full(w0i), full(b0), full(w1), full(b1),
            full(w2), full(b2), full(wpg), full(wpm), full(bp),
        ],
        out_specs=pl.BlockSpec((_BLK,), lambda i: (i,)),
        out_shape=jax.ShapeDtypeStruct((B,), jnp.float32),
    )(guw, giw, mu, mi, su, si, w0u, w0i, b0, w1, b1, w2, b2, wpg, wpm, bp)


def kernel(x, eu_gmf, ei_gmf, eu_mlp, ei_mlp, W0, b0, W1, b1, W2, b2, Wp, bp):
    uidx = x[:, 0]
    iidx = x[:, 1]
    n_pack = MLP_D // GMF_D
    eu_gmf_w = eu_gmf.reshape(eu_gmf.shape[0] // n_pack, MLP_D)
    ei_gmf_w = ei_gmf.reshape(ei_gmf.shape[0] // n_pack, MLP_D)
    guw, giw, mu, mi = _sc_gather_fn()(
        uidx.reshape(1, B), iidx.reshape(1, B),
        (uidx // n_pack).reshape(1, B), (iidx // n_pack).reshape(1, B),
        eu_gmf_w, ei_gmf_w, eu_mlp, ei_mlp)
    pred = _tc_fuse(
        guw, giw, mu, mi,
        (uidx % n_pack).reshape(B, 1), (iidx % n_pack).reshape(B, 1),
        W0[:, :MLP_D].T, W0[:, MLP_D:].T, b0.reshape(1, -1),
        W1.T, b1.reshape(1, -1), W2.T, b2.reshape(1, -1),
        Wp[:, :GMF_D], Wp[:, GMF_D:], bp.reshape(1, 1),
    )
    return pred


# speedup vs baseline: 1.6447x; 1.6382x over previous
import functools

import jax
import jax.numpy as jnp
from jax import lax
from jax.experimental import pallas as pl
from jax.experimental.pallas import tpu as pltpu
from jax.experimental.pallas import tpu_sc as plsc

B = 16384
GMF_D = 32
MLP_D = 128
_W = 128
_PACK_C = 4096


def _pack_body(euT, eiT, out):
    ii = (lax.broadcasted_iota(jnp.int32, (GMF_D, GMF_D), 0)
          == lax.broadcasted_iota(jnp.int32, (GMF_D, GMF_D), 1)
          ).astype(jnp.float32)
    dn = (((0,), (0,)), ((), ()))
    eu = lax.dot_general(euT[...], ii, dn, preferred_element_type=jnp.float32)
    ei = lax.dot_general(eiT[...], ii, dn, preferred_element_type=jnp.float32)
    out[:, 0:GMF_D] = eu
    out[:, GMF_D:2 * GMF_D] = ei
    out[:, 2 * GMF_D:] = jnp.zeros((out.shape[0], MLP_D - 2 * GMF_D),
                                   jnp.float32)


def _pack_gmf(eu_gmf_t, ei_gmf_t):
    n = eu_gmf_t.shape[1]
    grid = (pl.cdiv(n, _PACK_C),)
    return pl.pallas_call(
        _pack_body,
        grid=grid,
        in_specs=[
            pl.BlockSpec((GMF_D, _PACK_C), lambda i: (0, i)),
            pl.BlockSpec((GMF_D, _PACK_C), lambda i: (0, i)),
        ],
        out_specs=pl.BlockSpec((_PACK_C, MLP_D), lambda i: (i, 0)),
        out_shape=jax.ShapeDtypeStruct((n, MLP_D), jnp.float32),
    )(eu_gmf_t, ei_gmf_t)


@functools.cache
def _sc_mlp_gather_fn():
    mesh = plsc.VectorSubcoreMesh(core_axis_name="core",
                                  subcore_axis_name="subcore")

    @functools.partial(
        pl.kernel,
        out_type=(
            jax.ShapeDtypeStruct((B, MLP_D), jnp.float32),
            jax.ShapeDtypeStruct((B, MLP_D), jnp.float32),
        ),
        mesh=mesh,
        scratch_types=[pltpu.SemaphoreType.DMA] * 2,
    )
    def _sc_mlp(uidx_hbm, iidx_hbm, eu_mlp_hbm, ei_mlp_hbm, mu_hbm, mi_hbm,
                s0, s1):
        idx_spec = pl.BlockSpec((1, _W), lambda i: (0, i))
        row_spec = pl.BlockSpec((_W, MLP_D), lambda i: (i, 0))

        def body(u_v, i_v, mu_v, mi_v):
            c0 = pltpu.make_async_copy(eu_mlp_hbm.at[u_v.at[0]], mu_v, s0)
            c1 = pltpu.make_async_copy(ei_mlp_hbm.at[i_v.at[0]], mi_v, s1)
            c0.start(); c1.start()
            c0.wait(); c1.wait()

        pltpu.emit_pipeline(
            body,
            grid=(B // _W,),
            in_specs=[idx_spec, idx_spec],
            out_specs=[row_spec, row_spec],
            core_axis_name=("core", "subcore"),
            dimension_semantics=(pltpu.PARALLEL,),
        )(uidx_hbm, iidx_hbm, mu_hbm, mi_hbm)

    return _sc_mlp


@functools.cache
def _sc_gmf_gather_fn():
    mesh = plsc.VectorSubcoreMesh(core_axis_name="core",
                                  subcore_axis_name="subcore")

    @functools.partial(
        pl.kernel,
        out_type=jax.ShapeDtypeStruct((B, GMF_D), jnp.float32),
        mesh=mesh,
        scratch_types=[
            pltpu.VMEM((_W, MLP_D), jnp.float32),
            pltpu.VMEM((_W, MLP_D), jnp.float32),
            pltpu.SemaphoreType.DMA,
            pltpu.SemaphoreType.DMA,
        ],
    )
    def _sc_gmf(uidx_hbm, iidx_hbm, pack_hbm, g_hbm, wu_v, wi_v, s0, s1):
        idx_spec = pl.BlockSpec((1, _W), lambda i: (0, i))

        def body(u_v, i_v, g_v):
            c0 = pltpu.make_async_copy(pack_hbm.at[u_v.at[0]], wu_v, s0)
            c1 = pltpu.make_async_copy(pack_hbm.at[i_v.at[0]], wi_v, s1)
            c0.start(); c1.start()
            c0.wait(); c1.wait()

            @pl.loop(0, _W)
            def _(k):
                g_v[k, pl.ds(0, 16)] = (
                    wu_v[k, pl.ds(0, 16)] * wi_v[k, pl.ds(GMF_D, 16)])
                g_v[k, pl.ds(16, 16)] = (
                    wu_v[k, pl.ds(16, 16)] * wi_v[k, pl.ds(GMF_D + 16, 16)])

        pltpu.emit_pipeline(
            body,
            grid=(B // _W,),
            in_specs=[idx_spec, idx_spec],
            out_specs=[pl.BlockSpec((_W, GMF_D), lambda i: (i, 0))],
            core_axis_name=("core", "subcore"),
            dimension_semantics=(pltpu.PARALLEL,),
        )(uidx_hbm, iidx_hbm, g_hbm)

    return _sc_gmf


_BLK = 2048


def _tc_body(g, mu, mi, w0u, w0i, b0, w1, b1, w2, b2, wpg, wpm, bp, out):
    h = jnp.dot(mu[...].astype(jnp.bfloat16), w0u[...].astype(jnp.bfloat16),
                preferred_element_type=jnp.float32)
    h = h + jnp.dot(mi[...].astype(jnp.bfloat16),
                    w0i[...].astype(jnp.bfloat16),
                    preferred_element_type=jnp.float32)
    h = jnp.maximum(h + b0[...], 0.0)
    h = jnp.maximum(
        jnp.dot(h, w1[...], preferred_element_type=jnp.float32) + b1[...], 0.0)
    h = jnp.maximum(
        jnp.dot(h, w2[...], preferred_element_type=jnp.float32) + b2[...], 0.0)
    pg = jnp.dot(g[...], wpg[...], preferred_element_type=jnp.float32)
    pm = jnp.dot(h, wpm[...], preferred_element_type=jnp.float32)
    out[...] = pg[:, 0] + pm[:, 0] + bp[0, 0]


def _tc_fuse(g, mu, mi, w0u, w0i, b0, w1, b1, w2, b2, wpg, wpm, bp):
    n_blk = B // _BLK
    batch_spec = lambda d: pl.BlockSpec((_BLK, d), lambda i: (i, 0))
    full = lambda a: pl.BlockSpec(a.shape, lambda i: (0,) * a.ndim)
    return pl.pallas_call(
        _tc_body,
        grid=(n_blk,),
        in_specs=[
            batch_spec(GMF_D), batch_spec(MLP_D), batch_spec(MLP_D),
            full(w0u), full(w0i), full(b0), full(w1), full(b1),
            full(w2), full(b2), full(wpg), full(wpm), full(bp),
        ],
        out_specs=pl.BlockSpec((_BLK,), lambda i: (i,)),
        out_shape=jax.ShapeDtypeStruct((B,), jnp.float32),
    )(g, mu, mi, w0u, w0i, b0, w1, b1, w2, b2, wpg, wpm, bp)


def kernel(x, eu_gmf, ei_gmf, eu_mlp, ei_mlp, W0, b0, W1, b1, W2, b2, Wp, bp):
    uidx = x[:, 0].reshape(1, B)
    iidx = x[:, 1].reshape(1, B)
    mu, mi = _sc_mlp_gather_fn()(uidx, iidx, eu_mlp, ei_mlp)
    gmf_pack = _pack_gmf(eu_gmf.T, ei_gmf.T)
    g = _sc_gmf_gather_fn()(uidx, iidx, gmf_pack)
    pred = _tc_fuse(
        g, mu, mi,
        W0[:, :MLP_D].T, W0[:, MLP_D:].T, b0.reshape(1, -1),
        W1.T, b1.reshape(1, -1), W2.T, b2.reshape(1, -1),
        Wp[:, :GMF_D].T, Wp[:, GMF_D:].T, bp.reshape(1, 1),
    )
    return pred
